# baseline (speedup 1.0000x reference)
import functools

import jax
import jax.numpy as jnp
from jax import lax
from jax.experimental import pallas as pl
from jax.experimental.pallas import tpu as pltpu
from jax.experimental.pallas import tpu_sc as plsc

DM = 768
DFF = 3072
E = 64
NTOK = 4096
BLK = 2048
NBLK = NTOK // BLK
T = 128
TW = 128
NT_MAX = NTOK // T + E
NPAD = NT_MAX * T
FF_C = 3072
FH = DFF // 2
NFF = DFF // FF_C


def _router_body(x_ref, rw_ref, rb_ref, e_ref, coef_ref, rank_ref,
                 off_ref, te_ref, nt_ref, counts):
    i = pl.program_id(0)

    @pl.when(i == 0)
    def _():
        counts[...] = jnp.zeros((1, E), jnp.float32)

    xb = x_ref[...]
    logits = lax.dot_general(xb, rw_ref[...],
                             (((1,), (1,)), ((), ())),
                             preferred_element_type=jnp.float32)
    logits = logits + rb_ref[...]

    lmax = jnp.max(logits, axis=1, keepdims=True)
    denom = jnp.sum(jnp.exp(logits - lmax), axis=1, keepdims=True)
    w = 1.0 / denom
    coef = w / (w + 1e-9)

    ecols = lax.broadcasted_iota(jnp.int32, (BLK, E), 1).astype(jnp.float32)
    e_f = jnp.min(jnp.where(logits >= lmax, ecols, float(E)),
                  axis=1, keepdims=True)
    onehot = (ecols == e_f).astype(jnp.float32)

    r_i = lax.broadcasted_iota(jnp.int32, (BLK, BLK), 0)
    c_i = lax.broadcasted_iota(jnp.int32, (BLK, BLK), 1)
    tri = (r_i > c_i).astype(jnp.float32)
    pre = lax.dot_general(tri, onehot, (((1,), (0,)), ((), ())),
                          preferred_element_type=jnp.float32)
    rank_in = jnp.sum(pre * onehot, axis=1, keepdims=True)
    prev = jnp.sum(counts[...] * onehot, axis=1, keepdims=True)
    rank = prev + rank_in

    e_ref[...] = e_f.astype(jnp.int32)[None]
    coef_ref[...] = coef[None]
    rank_ref[...] = rank.astype(jnp.int32)[None]

    counts[...] = counts[...] + jnp.sum(onehot, axis=0, keepdims=True)

    @pl.when(i == NBLK - 1)
    def _():
        c_row = counts[...]
        nt_row = jnp.ceil(c_row / float(T))
        a_i = lax.broadcasted_iota(jnp.int32, (E, E), 0)
        b_i = lax.broadcasted_iota(jnp.int32, (E, E), 1)
        upper = (a_i <= b_i).astype(jnp.float32)
        cum = lax.dot_general(nt_row, upper, (((1,), (0,)), ((), ())),
                              preferred_element_type=jnp.float32)
        off_ref[...] = (float(T) * (cum - nt_row)).astype(jnp.int32)

        j_i = lax.broadcasted_iota(jnp.int32,
                                   (NT_MAX, E), 0).astype(jnp.float32)
        raw = jnp.sum((j_i >= cum).astype(jnp.float32), axis=1, keepdims=True)
        ntotal = jnp.max(cum)
        e_ids = lax.broadcasted_iota(jnp.int32, (1, E), 1).astype(jnp.float32)
        last_e = jnp.max(e_ids * (nt_row > 0.0).astype(jnp.float32))
        jcol = lax.broadcasted_iota(jnp.int32,
                                    (NT_MAX, 1), 0).astype(jnp.float32)
        te = jnp.where(jcol < ntotal, raw, last_e)
        te_ref[...] = te.astype(jnp.int32)
        nt_ref[...] = (jnp.zeros((1, 1), jnp.float32)
                       + ntotal).astype(jnp.int32)


def _router(x2, rw, rb, interpret=False):
    out = pl.pallas_call(
        _router_body,
        grid=(NBLK,),
        in_specs=[
            pl.BlockSpec((BLK, DM), lambda i: (i, 0)),
            pl.BlockSpec((E, DM), lambda i: (0, 0)),
            pl.BlockSpec((1, E), lambda i: (0, 0)),
        ],
        out_specs=[
            pl.BlockSpec((1, BLK, 1), lambda i: (i, 0, 0)),
            pl.BlockSpec((1, BLK, 1), lambda i: (i, 0, 0)),
            pl.BlockSpec((1, BLK, 1), lambda i: (i, 0, 0)),
            pl.BlockSpec((1, E), lambda i: (0, 0)),
            pl.BlockSpec((NT_MAX, 1), lambda i: (0, 0)),
            pl.BlockSpec((1, 1), lambda i: (0, 0)),
        ],
        out_shape=[
            jax.ShapeDtypeStruct((NBLK, BLK, 1), jnp.int32),
            jax.ShapeDtypeStruct((NBLK, BLK, 1), jnp.float32),
            jax.ShapeDtypeStruct((NBLK, BLK, 1), jnp.int32),
            jax.ShapeDtypeStruct((1, E), jnp.int32),
            jax.ShapeDtypeStruct((NT_MAX, 1), jnp.int32),
            jax.ShapeDtypeStruct((1, 1), jnp.int32),
        ],
        scratch_shapes=[pltpu.VMEM((1, E), jnp.float32)],
        compiler_params=pltpu.CompilerParams(
            dimension_semantics=("arbitrary",)),
        interpret=interpret,
    )(x2, rw, rb)
    return out


def _sc_wid():
    info = plsc.get_sparse_core_info()
    return lax.axis_index("s") * info.num_cores + lax.axis_index("c")


def _scatter_body(x_hbm, e_hbm, r_hbm, off_hbm, xs_hbm, dest_hbm,
                  e_v, r_v, off_v, idx_v, rows_v, sem):
    base = _sc_wid() * TW
    xcp = pltpu.async_copy(x_hbm.at[pl.ds(base, TW)], rows_v, sem)
    pltpu.sync_copy(e_hbm.at[pl.ds(base, TW)], e_v)
    pltpu.sync_copy(r_hbm.at[pl.ds(base, TW)], r_v)
    pltpu.sync_copy(off_hbm, off_v)
    for i in range(TW // 16):
        e16 = e_v[pl.ds(i * 16, 16)]
        og = plsc.load_gather(off_v, [e16])
        idx_v[pl.ds(i * 16, 16)] = og + r_v[pl.ds(i * 16, 16)]
    xcp.wait()
    pltpu.async_copy(rows_v, xs_hbm.at[idx_v], sem).wait()
    pltpu.sync_copy(idx_v, dest_hbm.at[pl.ds(base, TW)])


@functools.cache
def _scatter():
    return pl.kernel(
        _scatter_body,
        mesh=plsc.VectorSubcoreMesh(core_axis_name="c", subcore_axis_name="s"),
        out_type=[jax.ShapeDtypeStruct((NPAD, DM), jnp.float32),
                  jax.ShapeDtypeStruct((NTOK,), jnp.int32)],
        scratch_types=[pltpu.VMEM((TW,), jnp.int32),
                       pltpu.VMEM((TW,), jnp.int32),
                       pltpu.VMEM((E,), jnp.int32),
                       pltpu.VMEM((TW,), jnp.int32),
                       pltpu.VMEM((TW, DM), jnp.float32),
                       pltpu.SemaphoreType.DMA],
        compiler_params=pltpu.CompilerParams(needs_layout_passes=False))


def _ffn_body(te_ref, nt_ref, x_ref, w1a_ref, w1b_ref, b1_ref,
              w2a_ref, w2b_ref, b2_ref, out_ref):
    i = pl.program_id(0)

    @pl.when(i < nt_ref[0])
    def _():
        xb = x_ref[...]

        def half_ffn(w1h_ref, w2h_ref, lo):
            h = lax.dot_general(xb, w1h_ref[0],
                                (((1,), (1,)), ((), ())),
                                preferred_element_type=jnp.float32)
            h = h + b1_ref[0, :, lo:lo + FH]
            h = 0.5 * h * (1.0 + lax.erf(h * (2.0 ** -0.5)))
            return lax.dot_general(h, w2h_ref[0],
                                   (((1,), (1,)), ((), ())),
                                   preferred_element_type=jnp.float32)

        y = half_ffn(w1a_ref, w2a_ref, 0) + half_ffn(w1b_ref, w2b_ref, FH)
        out_ref[...] = y + b2_ref[0]


def _ffn(te, nt, xs, w1, b1r, w2, b2r, interpret=False):
    grid_spec = pltpu.PrefetchScalarGridSpec(
        num_scalar_prefetch=2,
        grid=(NT_MAX,),
        in_specs=[
            pl.BlockSpec((T, DM),
                         lambda i, te, nt: (jnp.minimum(i, nt[0] - 1), 0)),
            pl.BlockSpec((1, FH, DM), lambda i, te, nt: (te[i], 0, 0)),
            pl.BlockSpec((1, FH, DM), lambda i, te, nt: (te[i], 1, 0)),
            pl.BlockSpec((1, 1, DFF), lambda i, te, nt: (te[i], 0, 0)),
            pl.BlockSpec((1, DM, FH), lambda i, te, nt: (te[i], 0, 0)),
            pl.BlockSpec((1, DM, FH), lambda i, te, nt: (te[i], 0, 1)),
            pl.BlockSpec((1, 1, DM), lambda i, te, nt: (te[i], 0, 0)),
        ],
        out_specs=pl.BlockSpec(
            (T, DM), lambda i, te, nt: (jnp.minimum(i, nt[0] - 1), 0)),
        scratch_shapes=[],
    )
    return pl.pallas_call(
        _ffn_body,
        grid_spec=grid_spec,
        out_shape=jax.ShapeDtypeStruct((NPAD, DM), jnp.float32),
        compiler_params=pltpu.CompilerParams(
            dimension_semantics=("arbitrary",)),
        interpret=interpret,
    )(te, nt, xs, w1, w1, b1r, w2, w2, b2r)


def _gather_body(ys_hbm, dest_hbm, coef_hbm, y_hbm, idx_v, c_v, rows_v,
                 sem, sem2):
    base = _sc_wid() * TW
    half = TW // 2
    pltpu.sync_copy(dest_hbm.at[pl.ds(base, TW)], idx_v)
    pltpu.sync_copy(coef_hbm.at[pl.ds(base, TW)], c_v)
    cp0 = pltpu.async_copy(ys_hbm.at[idx_v.at[pl.ds(0, half)]],
                           rows_v.at[pl.ds(0, half)], sem)
    cp1 = pltpu.async_copy(ys_hbm.at[idx_v.at[pl.ds(half, half)]],
                           rows_v.at[pl.ds(half, half)], sem2)

    def scale(lo):
        def body(r, carry):
            cg = plsc.load_gather(c_v, [jnp.zeros((16,), jnp.int32) + r])
            for k in range(DM // 16):
                sl = pl.ds(k * 16, 16)
                rows_v[r, sl] = rows_v[r, sl] * cg
            return carry
        lax.fori_loop(lo, lo + half, body, 0)

    cp0.wait()
    scale(0)
    ocp0 = pltpu.async_copy(rows_v.at[pl.ds(0, half)],
                            y_hbm.at[pl.ds(base, half)], sem)
    cp1.wait()
    scale(half)
    ocp0.wait()
    pltpu.sync_copy(rows_v.at[pl.ds(half, half)],
                    y_hbm.at[pl.ds(base + half, half)])


@functools.cache
def _gather():
    return pl.kernel(
        _gather_body,
        mesh=plsc.VectorSubcoreMesh(core_axis_name="c", subcore_axis_name="s"),
        out_type=jax.ShapeDtypeStruct((NTOK, DM), jnp.float32),
        scratch_types=[pltpu.VMEM((TW,), jnp.int32),
                       pltpu.VMEM((TW,), jnp.float32),
                       pltpu.VMEM((TW, DM), jnp.float32),
                       pltpu.SemaphoreType.DMA,
                       pltpu.SemaphoreType.DMA],
        compiler_params=pltpu.CompilerParams(needs_layout_passes=False))


def kernel(x, router_w, router_b, w1, b1, w2, b2):
    x2 = x.reshape(NTOK, DM)
    e3, coef3, r3, off, te, nt = _router(x2, router_w,
                                         router_b.reshape(1, E))
    expert = e3.reshape(NTOK)
    coef = coef3.reshape(NTOK)
    rank = r3.reshape(NTOK)

    xs, dest = _scatter()(x2, expert, rank, off.reshape(E))
    ys = _ffn(te.reshape(NT_MAX), nt.reshape(1), xs,
              w1, b1.reshape(E, 1, DFF), w2, b2.reshape(E, 1, DM))
    y = _gather()(ys, dest, coef)
    return y.reshape(1, NTOK, DM)


# speedup vs baseline: 1.0062x; 1.0062x over previous
import functools

import jax
import jax.numpy as jnp
from jax import lax
from jax.experimental import pallas as pl
from jax.experimental.pallas import tpu as pltpu
from jax.experimental.pallas import tpu_sc as plsc

DM = 768
DFF = 3072
E = 64
NTOK = 4096
BLK = 1024
NBLK = NTOK // BLK
T = 128
TW = 128
NT_MAX = NTOK // T + E
NPAD = NT_MAX * T
FH = DFF // 2


def _router_body(x_ref, rw_ref, rb_ref, e_ref, coef_ref, rank_ref,
                 off_ref, te_ref, nt_ref, counts):
    i = pl.program_id(0)

    @pl.when(i == 0)
    def _():
        counts[...] = jnp.zeros((1, E), jnp.float32)

    xb = x_ref[...]
    logits = lax.dot_general(xb, rw_ref[...],
                             (((1,), (1,)), ((), ())),
                             preferred_element_type=jnp.float32)
    logits = logits + rb_ref[...]

    lmax = jnp.max(logits, axis=1, keepdims=True)
    denom = jnp.sum(jnp.exp(logits - lmax), axis=1, keepdims=True)
    w = 1.0 / denom
    coef = w / (w + 1e-9)

    ecols = lax.broadcasted_iota(jnp.int32, (BLK, E), 1).astype(jnp.float32)
    e_f = jnp.min(jnp.where(logits >= lmax, ecols, float(E)),
                  axis=1, keepdims=True)
    onehot = (ecols == e_f).astype(jnp.float32)

    r_i = lax.broadcasted_iota(jnp.int32, (BLK, BLK), 0)
    c_i = lax.broadcasted_iota(jnp.int32, (BLK, BLK), 1)
    tri = (r_i > c_i).astype(jnp.float32)
    pre = lax.dot_general(tri, onehot, (((1,), (0,)), ((), ())),
                          preferred_element_type=jnp.float32)
    rank_in = jnp.sum(pre * onehot, axis=1, keepdims=True)
    prev = jnp.sum(counts[...] * onehot, axis=1, keepdims=True)
    rank = prev + rank_in

    e_ref[...] = e_f.astype(jnp.int32)[None]
    coef_ref[...] = coef[None]
    rank_ref[...] = rank.astype(jnp.int32)[None]

    counts[...] = counts[...] + jnp.sum(onehot, axis=0, keepdims=True)

    @pl.when(i == NBLK - 1)
    def _():
        c_row = counts[...]
        nt_row = jnp.ceil(c_row / float(T))
        a_i = lax.broadcasted_iota(jnp.int32, (E, E), 0)
        b_i = lax.broadcasted_iota(jnp.int32, (E, E), 1)
        upper = (a_i <= b_i).astype(jnp.float32)
        cum = lax.dot_general(nt_row, upper, (((1,), (0,)), ((), ())),
                              preferred_element_type=jnp.float32)
        off_ref[...] = (float(T) * (cum - nt_row)).astype(jnp.int32)

        j_i = lax.broadcasted_iota(jnp.int32,
                                   (NT_MAX, E), 0).astype(jnp.float32)
        raw = jnp.sum((j_i >= cum).astype(jnp.float32), axis=1, keepdims=True)
        ntotal = jnp.max(cum)
        e_ids = lax.broadcasted_iota(jnp.int32, (1, E), 1).astype(jnp.float32)
        last_e = jnp.max(e_ids * (nt_row > 0.0).astype(jnp.float32))
        jcol = lax.broadcasted_iota(jnp.int32,
                                    (NT_MAX, 1), 0).astype(jnp.float32)
        te = jnp.where(jcol < ntotal, raw, last_e)
        te_ref[...] = te.astype(jnp.int32)
        nt_ref[...] = (jnp.zeros((1, 1), jnp.float32)
                       + ntotal).astype(jnp.int32)


def _router(x2, rw, rb, interpret=False):
    out = pl.pallas_call(
        _router_body,
        grid=(NBLK,),
        in_specs=[
            pl.BlockSpec((BLK, DM), lambda i: (i, 0)),
            pl.BlockSpec((E, DM), lambda i: (0, 0)),
            pl.BlockSpec((1, E), lambda i: (0, 0)),
        ],
        out_specs=[
            pl.BlockSpec((1, BLK, 1), lambda i: (i, 0, 0)),
            pl.BlockSpec((1, BLK, 1), lambda i: (i, 0, 0)),
            pl.BlockSpec((1, BLK, 1), lambda i: (i, 0, 0)),
            pl.BlockSpec((1, E), lambda i: (0, 0)),
            pl.BlockSpec((NT_MAX, 1), lambda i: (0, 0)),
            pl.BlockSpec((1, 1), lambda i: (0, 0)),
        ],
        out_shape=[
            jax.ShapeDtypeStruct((NBLK, BLK, 1), jnp.int32),
            jax.ShapeDtypeStruct((NBLK, BLK, 1), jnp.float32),
            jax.ShapeDtypeStruct((NBLK, BLK, 1), jnp.int32),
            jax.ShapeDtypeStruct((1, E), jnp.int32),
            jax.ShapeDtypeStruct((NT_MAX, 1), jnp.int32),
            jax.ShapeDtypeStruct((1, 1), jnp.int32),
        ],
        scratch_shapes=[pltpu.VMEM((1, E), jnp.float32)],
        compiler_params=pltpu.CompilerParams(
            dimension_semantics=("arbitrary",)),
        interpret=interpret,
    )(x2, rw, rb)
    return out


def _sc_wid():
    info = plsc.get_sparse_core_info()
    return lax.axis_index("s") * info.num_cores + lax.axis_index("c")


def _scatter_body(x_hbm, e_hbm, r_hbm, off_hbm, xs_hbm, dest_hbm,
                  e_v, r_v, off_v, idx_v, rows_v, sem):
    base = _sc_wid() * TW
    xcp = pltpu.async_copy(x_hbm.at[pl.ds(base, TW)], rows_v, sem)
    pltpu.sync_copy(e_hbm.at[pl.ds(base, TW)], e_v)
    pltpu.sync_copy(r_hbm.at[pl.ds(base, TW)], r_v)
    pltpu.sync_copy(off_hbm, off_v)
    for i in range(TW // 16):
        e16 = e_v[pl.ds(i * 16, 16)]
        og = plsc.load_gather(off_v, [e16])
        idx_v[pl.ds(i * 16, 16)] = og + r_v[pl.ds(i * 16, 16)]
    xcp.wait()
    pltpu.async_copy(rows_v, xs_hbm.at[idx_v], sem).wait()
    pltpu.sync_copy(idx_v, dest_hbm.at[pl.ds(base, TW)])


@functools.cache
def _scatter():
    return pl.kernel(
        _scatter_body,
        mesh=plsc.VectorSubcoreMesh(core_axis_name="c", subcore_axis_name="s"),
        out_type=[jax.ShapeDtypeStruct((NPAD, DM), jnp.float32),
                  jax.ShapeDtypeStruct((NTOK,), jnp.int32)],
        scratch_types=[pltpu.VMEM((TW,), jnp.int32),
                       pltpu.VMEM((TW,), jnp.int32),
                       pltpu.VMEM((E,), jnp.int32),
                       pltpu.VMEM((TW,), jnp.int32),
                       pltpu.VMEM((TW, DM), jnp.float32),
                       pltpu.SemaphoreType.DMA],
        compiler_params=pltpu.CompilerParams(needs_layout_passes=False))


def _ffn_body(te_ref, nt_ref, x_ref, w1a_ref, w1b_ref, b1_ref,
              w2a_ref, w2b_ref, b2_ref, out_ref):
    i = pl.program_id(0)

    @pl.when(i < nt_ref[0])
    def _():
        xb = x_ref[...]

        def half_ffn(w1h_ref, w2h_ref, lo):
            h = lax.dot_general(xb, w1h_ref[0],
                                (((1,), (1,)), ((), ())),
                                preferred_element_type=jnp.float32)
            h = h + b1_ref[0, :, lo:lo + FH]
            h = 0.5 * h * (1.0 + lax.erf(h * (2.0 ** -0.5)))
            return lax.dot_general(h, w2h_ref[0],
                                   (((1,), (1,)), ((), ())),
                                   preferred_element_type=jnp.float32)

        y = half_ffn(w1a_ref, w2a_ref, 0) + half_ffn(w1b_ref, w2b_ref, FH)
        out_ref[...] = y + b2_ref[0]


def _ffn(te, nt, xs, w1, b1r, w2, b2r, interpret=False):
    grid_spec = pltpu.PrefetchScalarGridSpec(
        num_scalar_prefetch=2,
        grid=(NT_MAX,),
        in_specs=[
            pl.BlockSpec((T, DM),
                         lambda i, te, nt: (jnp.minimum(i, nt[0] - 1), 0)),
            pl.BlockSpec((1, FH, DM), lambda i, te, nt: (te[i], 0, 0)),
            pl.BlockSpec((1, FH, DM), lambda i, te, nt: (te[i], 1, 0)),
            pl.BlockSpec((1, 1, DFF), lambda i, te, nt: (te[i], 0, 0)),
            pl.BlockSpec((1, DM, FH), lambda i, te, nt: (te[i], 0, 0)),
            pl.BlockSpec((1, DM, FH), lambda i, te, nt: (te[i], 0, 1)),
            pl.BlockSpec((1, 1, DM), lambda i, te, nt: (te[i], 0, 0)),
        ],
        out_specs=pl.BlockSpec(
            (T, DM), lambda i, te, nt: (jnp.minimum(i, nt[0] - 1), 0)),
        scratch_shapes=[],
    )
    return pl.pallas_call(
        _ffn_body,
        grid_spec=grid_spec,
        out_shape=jax.ShapeDtypeStruct((NPAD, DM), jnp.float32),
        compiler_params=pltpu.CompilerParams(
            dimension_semantics=("arbitrary",)),
        interpret=interpret,
    )(te, nt, xs, w1, w1, b1r, w2, w2, b2r)


def _gather_body(ys_hbm, dest_hbm, coef_hbm, y_hbm, idx_v, c_v, rows_v,
                 sem, sem2):
    base = _sc_wid() * TW
    half = TW // 2
    pltpu.sync_copy(dest_hbm.at[pl.ds(base, TW)], idx_v)
    pltpu.sync_copy(coef_hbm.at[pl.ds(base, TW)], c_v)
    cp0 = pltpu.async_copy(ys_hbm.at[idx_v.at[pl.ds(0, half)]],
                           rows_v.at[pl.ds(0, half)], sem)
    cp1 = pltpu.async_copy(ys_hbm.at[idx_v.at[pl.ds(half, half)]],
                           rows_v.at[pl.ds(half, half)], sem2)

    def scale(lo):
        def body(r, carry):
            cg = plsc.load_gather(c_v, [jnp.zeros((16,), jnp.int32) + r])
            for k in range(DM // 16):
                sl = pl.ds(k * 16, 16)
                rows_v[r, sl] = rows_v[r, sl] * cg
            return carry
        lax.fori_loop(lo, lo + half, body, 0)

    cp0.wait()
    scale(0)
    ocp0 = pltpu.async_copy(rows_v.at[pl.ds(0, half)],
                            y_hbm.at[pl.ds(base, half)], sem)
    cp1.wait()
    scale(half)
    ocp0.wait()
    pltpu.sync_copy(rows_v.at[pl.ds(half, half)],
                    y_hbm.at[pl.ds(base + half, half)])


@functools.cache
def _gather():
    return pl.kernel(
        _gather_body,
        mesh=plsc.VectorSubcoreMesh(core_axis_name="c", subcore_axis_name="s"),
        out_type=jax.ShapeDtypeStruct((NTOK, DM), jnp.float32),
        scratch_types=[pltpu.VMEM((TW,), jnp.int32),
                       pltpu.VMEM((TW,), jnp.float32),
                       pltpu.VMEM((TW, DM), jnp.float32),
                       pltpu.SemaphoreType.DMA,
                       pltpu.SemaphoreType.DMA],
        compiler_params=pltpu.CompilerParams(needs_layout_passes=False))


def kernel(x, router_w, router_b, w1, b1, w2, b2):
    x2 = x.reshape(NTOK, DM)
    e3, coef3, r3, off, te, nt = _router(x2, router_w,
                                         router_b.reshape(1, E))
    expert = e3.reshape(NTOK)
    coef = coef3.reshape(NTOK)
    rank = r3.reshape(NTOK)

    xs, dest = _scatter()(x2, expert, rank, off.reshape(E))
    ys = _ffn(te.reshape(NT_MAX), nt.reshape(1), xs,
              w1, b1.reshape(E, 1, DFF), w2, b2.reshape(E, 1, DM))
    y = _gather()(ys, dest, coef)
    return y.reshape(1, NTOK, DM)
